# baseline (speedup 1.0000x reference)
import functools
import math

import jax
import jax.numpy as jnp
import numpy as np
from jax import lax
from jax.experimental import pallas as pl
from jax.experimental.pallas import tpu as pltpu
from jax.experimental.pallas import tpu_sc as plsc

VOCAB = 100000
MAX_TOKENS = 2048
D_MODEL = 768
SCALE = math.sqrt(float(D_MODEL))

NUM_CORES = 2
NUM_SUBCORES = 16
NUM_WORKERS = NUM_CORES * NUM_SUBCORES

B, T = 4, MAX_TOKENS
TOTAL = B * T
POS_PER_W = T // NUM_WORKERS
NCHUNK = 8
POS_PER_CHUNK = POS_PER_W // NCHUNK
ROWS_PER_CHUNK = POS_PER_CHUNK * B
LANES = 16
PAIRS_PER_ROW = D_MODEL // (2 * LANES)
NBUF = 4
NGROUPS = NCHUNK // NBUF


def _pe_table_packed() -> np.ndarray:
    positions = np.arange(MAX_TOKENS)[:, np.newaxis]
    d_half = D_MODEL // 2
    d_scales = (1.0 / 10000 ** (np.arange(d_half) / d_half))[np.newaxis, :]
    pe = np.empty((MAX_TOKENS, D_MODEL), dtype=np.float32)
    pe[:, 0::2] = np.sin(positions * d_scales)
    pe[:, 1::2] = np.cos(positions * d_scales)
    bits = pe.view(np.uint32)
    bf_hi = ((bits + 0x8000 + ((bits >> 16) & 1)) >> 16).astype(np.uint32)
    g = bf_hi.reshape(MAX_TOKENS, D_MODEL // 32, 2, 16)
    packed = g[:, :, 0, :] | (g[:, :, 1, :] << 16)
    return packed.reshape(MAX_TOKENS, D_MODEL // 2).view(np.int32)


_PE = _pe_table_packed()

_mesh = plsc.VectorSubcoreMesh(
    core_axis_name="c",
    subcore_axis_name="s",
    num_cores=NUM_CORES,
    num_subcores=NUM_SUBCORES,
)


@functools.partial(
    pl.kernel,
    out_type=jax.ShapeDtypeStruct((TOTAL, D_MODEL), jnp.float32),
    mesh=_mesh,
    scratch_types=[
        pltpu.VMEM((B, POS_PER_W), jnp.int32),
        pltpu.VMEM((POS_PER_W, D_MODEL // 2), jnp.int32),
    ]
    + [pltpu.VMEM((ROWS_PER_CHUNK, D_MODEL), jnp.float32)] * NBUF
    + [pltpu.SemaphoreType.DMA] * (2 * NBUF),
)
def _emb_kernel(xf_hbm, table_hbm, pe_hbm, out_hbm, idx_v, pe_v, *bufs_sems):
    rbufs = bufs_sems[:NBUF]
    gsems = bufs_sems[NBUF:2 * NBUF]
    wsems = bufs_sems[2 * NBUF:]
    wid = lax.axis_index("s") * NUM_CORES + lax.axis_index("c")
    pos0 = wid * POS_PER_W

    idescs = [
        pltpu.make_async_copy(
            xf_hbm.at[b, pl.ds(pos0, POS_PER_W)], idx_v.at[b], gsems[0])
        for b in range(B)
    ]
    for d in idescs:
        d.start()
    for d in idescs:
        d.wait()

    def _gather(c, k):
        descs = [
            pltpu.make_async_copy(
                table_hbm.at[idx_v.at[b, pl.ds(c * POS_PER_CHUNK,
                                               POS_PER_CHUNK)]],
                rbufs[k].at[pl.ds(b * POS_PER_CHUNK, POS_PER_CHUNK)],
                gsems[k])
            for b in range(B)
        ]
        for d in descs:
            d.start()
        return descs

    for k in range(NBUF):
        _gather(k, k)
    pltpu.sync_copy(pe_hbm.at[pl.ds(pos0, POS_PER_W)], pe_v)

    def _wb_descs(k, c):
        rbuf = rbufs[k]
        descs = []
        for b in range(B):
            dst = out_hbm.at[
                pl.ds(b * T + pos0 + c * POS_PER_CHUNK, POS_PER_CHUNK)]
            src = rbuf.at[pl.ds(b * POS_PER_CHUNK, POS_PER_CHUNK)]
            descs.append(pltpu.make_async_copy(src, dst, wsems[k]))
        return descs

    def _group(g, carry):
        cbase = g * NBUF
        for k in range(NBUF):
            c = cbase + k
            rbuf = rbufs[k]
            for b in range(B):
                pltpu.make_async_copy(
                    table_hbm.at[idx_v.at[b, pl.ds(c * POS_PER_CHUNK,
                                                   POS_PER_CHUNK)]],
                    rbuf.at[pl.ds(b * POS_PER_CHUNK, POS_PER_CHUNK)],
                    gsems[k]).wait()

            def _fma(r, inner):
                prow = c * POS_PER_CHUNK + r
                for j0 in range(0, PAIRS_PER_ROW, 8):
                    pks = [pe_v[prow, pl.ds((j0 + u) * LANES, LANES)]
                           for u in range(8)]
                    pas = [lax.bitcast_convert_type(
                        lax.shift_left(pk, 16), jnp.float32) for pk in pks]
                    pbs = [lax.bitcast_convert_type(
                        lax.bitwise_and(pk, jnp.int32(-65536)), jnp.float32)
                        for pk in pks]
                    for b in range(B):
                        row = b * POS_PER_CHUNK + r
                        for u in range(8):
                            sa = pl.ds((j0 + u) * 2 * LANES, LANES)
                            sb = pl.ds((j0 + u) * 2 * LANES + LANES, LANES)
                            rbuf[row, sa] = rbuf[row, sa] * SCALE + pas[u]
                            rbuf[row, sb] = rbuf[row, sb] * SCALE + pbs[u]
                return inner

            lax.fori_loop(0, POS_PER_CHUNK, _fma, 0)
            for d in _wb_descs(k, c):
                d.start()

            j = (k - 1) % NBUF
            nxt = c - 1 + NBUF

            @pl.when(jnp.logical_and(c >= 1, nxt < NCHUNK))
            def _refill():
                for d in _wb_descs(j, c):
                    d.wait()
                _gather(nxt, j)

        return carry

    lax.fori_loop(0, NGROUPS, _group, 0)
    for k in range(NBUF):
        for d in _wb_descs(k, NCHUNK - NBUF + k):
            d.wait()


def kernel(x, table):
    out = _emb_kernel(x, table, _PE)
    return out.reshape(B, T, D_MODEL)
